# baseline (speedup 1.0000x reference)
import functools

import jax
import jax.numpy as jnp
from jax import lax
from jax.experimental import pallas as pl
from jax.experimental.pallas import tpu as pltpu
from jax.experimental.pallas import tpu_sc as plsc

N_NODES = 10000
N_EDGES = 160000
D_IN = 256
D_OUT = 256

NC = 2
NS = 16
Q = 64
CW = 16

BLK = 128
EPT = 10240
NBLK = EPT // BLK
NPAIR = NBLK // 2
EDGES_PAD = EPT * NS
N_GARB = 240
NACC = N_NODES + N_GARB

ROWS_PER_TILE = NACC // NS
ZROWS = 160


def _sc_segment_sum(x4, srcs, dsts):
    mesh = plsc.VectorSubcoreMesh(core_axis_name="c", subcore_axis_name="s")

    @functools.partial(
        pl.kernel,
        out_type=(jax.ShapeDtypeStruct((NC, NACC, 2 * Q), jnp.float32),
                  jax.ShapeDtypeStruct((NACC, CW), jnp.float32)),
        mesh=mesh,
        scratch_types=[
            pltpu.VMEM((NBLK, BLK), jnp.int32),
            pltpu.VMEM((NBLK, BLK), jnp.int32),
            pltpu.VMEM((NBLK, BLK), jnp.int32),
            pltpu.VMEM((BLK, Q), jnp.float32),
            pltpu.VMEM((BLK, Q), jnp.float32),
            pltpu.VMEM((ZROWS, Q), jnp.float32),
            pltpu.VMEM((ZROWS, CW), jnp.float32),
            pltpu.VMEM((BLK, CW), jnp.float32),
            pltpu.VMEM_SHARED((NACC, Q), jnp.float32),
            pltpu.VMEM_SHARED((NACC, CW), jnp.float32),
            pltpu.SemaphoreType.DMA,
            pltpu.SemaphoreType.DMA,
            pltpu.SemaphoreType.DMA,
            pltpu.SemaphoreType.DMA,
            pltpu.SemaphoreType.DMA,
            pltpu.SemaphoreType.DMA,
        ],
        compiler_params=pltpu.CompilerParams(use_tc_tiling_on_sc=False,
                                             vmem_limit_bytes=2 * 1024 * 1024),
    )
    def seg_sum(x4_hbm, srcs_hbm, dsts_hbm, out_hbm, outc_hbm,
                sidx0, sidx1, didx, buf0, buf1, zbuf, zbufc, obuf,
                acc, cacc, g0, g1, s0, s1, sc, sz):
        cid = lax.axis_index("c")
        tid = lax.axis_index("s")

        pltpu.async_copy(srcs_hbm.at[cid, 0, tid], sidx0, sz)
        pltpu.async_copy(srcs_hbm.at[cid, 1, tid], sidx1, sz)
        pltpu.async_copy(dsts_hbm.at[tid], didx, sz)

        zero16 = jnp.zeros((16,), jnp.float32)
        one16 = jnp.ones((16,), jnp.float32)

        def fill_zeros(i, _):
            for j in range(Q // 16):
                zbuf[i, pl.ds(j * 16, 16)] = zero16
            zbufc[i, pl.ds(0, 16)] = zero16
            return 0
        lax.fori_loop(0, ZROWS, fill_zeros, 0)

        def fill_ones(i, _):
            obuf[i, pl.ds(0, 16)] = one16
            return 0
        lax.fori_loop(0, BLK, fill_ones, 0)

        pltpu.make_async_copy(srcs_hbm.at[cid, 0, tid], sidx0, sz).wait()
        pltpu.make_async_copy(srcs_hbm.at[cid, 1, tid], sidx1, sz).wait()
        pltpu.make_async_copy(dsts_hbm.at[tid], didx, sz).wait()

        def zero_acc():
            for i in range(ROWS_PER_TILE // ZROWS):
                pltpu.async_copy(
                    zbuf, acc.at[pl.ds(tid * ROWS_PER_TILE + i * ZROWS, ZROWS)],
                    sz)
            for i in range(ROWS_PER_TILE // ZROWS):
                pltpu.make_async_copy(
                    zbuf, acc.at[pl.ds(0, ZROWS)], sz).wait()

        def run_pass(sidx, lane_off, with_counts):
            def start_gather(j, buf, sem):
                pltpu.async_copy(x4_hbm.at[sidx.at[j]], buf, sem)

            def wait_gather(buf, sem):
                pltpu.make_async_copy(x4_hbm.at[pl.ds(0, BLK)], buf, sem).wait()

            def start_scatter(j, buf, sem):
                pltpu.async_copy(buf, acc.at[didx.at[j]], sem, add=True)

            def wait_scatter(buf, sem):
                pltpu.make_async_copy(buf, acc.at[pl.ds(0, BLK)], sem).wait()

            def count_block(j):
                if with_counts:
                    @pl.when(cid == 0)
                    def _():
                        pltpu.async_copy(obuf, cacc.at[didx.at[j]], sc,
                                         add=True)

            def count_drain():
                if with_counts:
                    @pl.when(cid == 0)
                    def _():
                        pltpu.make_async_copy(
                            obuf, cacc.at[pl.ds(0, BLK)], sc).wait()

            start_gather(0, buf0, g0)

            def pair(i, _):
                wait_gather(buf0, g0)

                @pl.when(i > 0)
                def _():
                    wait_scatter(buf1, s1)
                    count_drain()
                    count_drain()

                start_gather(2 * i + 1, buf1, g1)
                start_scatter(2 * i, buf0, s0)
                count_block(2 * i)
                wait_gather(buf1, g1)
                wait_scatter(buf0, s0)

                @pl.when(i < NPAIR - 1)
                def _():
                    start_gather(2 * i + 2, buf0, g0)

                start_scatter(2 * i + 1, buf1, s1)
                count_block(2 * i + 1)
                return 0

            lax.fori_loop(0, NPAIR, pair, 0)
            wait_scatter(buf1, s1)
            count_drain()
            count_drain()
            plsc.subcore_barrier()

            pltpu.sync_copy(
                acc.at[pl.ds(tid * ROWS_PER_TILE, ROWS_PER_TILE)],
                out_hbm.at[cid, pl.ds(tid * ROWS_PER_TILE, ROWS_PER_TILE),
                           pl.ds(lane_off, Q)])

        zero_acc()

        @pl.when(cid == 0)
        def _():
            for i in range(ROWS_PER_TILE // ZROWS):
                pltpu.async_copy(
                    zbufc,
                    cacc.at[pl.ds(tid * ROWS_PER_TILE + i * ZROWS, ZROWS)],
                    sz)
            for i in range(ROWS_PER_TILE // ZROWS):
                pltpu.make_async_copy(
                    zbufc, cacc.at[pl.ds(0, ZROWS)], sz).wait()

        plsc.subcore_barrier()
        run_pass(sidx0, 0, with_counts=True)

        @pl.when(cid == 0)
        def _():
            pltpu.sync_copy(
                cacc.at[pl.ds(tid * ROWS_PER_TILE, ROWS_PER_TILE)],
                outc_hbm.at[pl.ds(tid * ROWS_PER_TILE, ROWS_PER_TILE)])

        zero_acc()
        plsc.subcore_barrier()
        run_pass(sidx1, Q, with_counts=False)

    return seg_sum(x4, srcs, dsts)


def _tc_linear(x, sums, cnt, W, b2, block_m=1000):
    m_blocks = N_NODES // block_m

    def body(x_ref, s0_ref, s1_ref, c_ref, w_ref, b_ref, o_ref):
        cnt_col = jnp.maximum(c_ref[:, 0:1], 1.0)
        agg = jnp.concatenate([s0_ref[0], s1_ref[0]], axis=1) / cnt_col
        w = w_ref[...]
        dn = (((1,), (1,)), ((), ()))
        acc = lax.dot_general(x_ref[...], w[:, :D_IN], dn,
                              preferred_element_type=jnp.float32)
        acc += lax.dot_general(agg, w[:, D_IN:], dn,
                               preferred_element_type=jnp.float32)
        o_ref[...] = acc + b_ref[...]

    return pl.pallas_call(
        body,
        grid=(m_blocks,),
        in_specs=[
            pl.BlockSpec((block_m, D_IN), lambda m: (m, 0)),
            pl.BlockSpec((1, block_m, 2 * Q), lambda m: (0, m, 0)),
            pl.BlockSpec((1, block_m, 2 * Q), lambda m: (1, m, 0)),
            pl.BlockSpec((block_m, CW), lambda m: (m, 0)),
            pl.BlockSpec((D_OUT, 2 * D_IN), lambda m: (0, 0)),
            pl.BlockSpec((1, D_OUT), lambda m: (0, 0)),
        ],
        out_specs=pl.BlockSpec((block_m, D_OUT), lambda m: (m, 0)),
        out_shape=jax.ShapeDtypeStruct((N_NODES, D_OUT), jnp.float32),
        compiler_params=pltpu.CompilerParams(
            dimension_semantics=("arbitrary",)),
    )(x, sums, sums, cnt, W, b2)


def kernel(x, edge_index, W, b):
    src = edge_index[0].astype(jnp.int32)
    dst = edge_index[1].astype(jnp.int32)

    npad = EDGES_PAD - N_EDGES
    pad_ids = lax.iota(jnp.int32, npad)
    psrc = jnp.concatenate([src, pad_ids])
    pdst = jnp.concatenate([dst, N_NODES + (pad_ids & 127)])
    psrc4 = psrc * 4
    srcs = jnp.stack([psrc4, psrc4 + 1, psrc4 + 2, psrc4 + 3])
    srcs = srcs.reshape(NC, 2, EDGES_PAD).reshape(NC, 2, NS, NBLK, BLK)
    dsts = pdst.reshape(NS, NBLK, BLK)

    x4 = x.reshape(4 * N_NODES, Q)

    sums, cnt = _sc_segment_sum(x4, srcs, dsts)
    return _tc_linear(x, sums, cnt, W, b.reshape(1, D_OUT))


# speedup vs baseline: 1.0089x; 1.0089x over previous
import functools

import jax
import jax.numpy as jnp
from jax import lax
from jax.experimental import pallas as pl
from jax.experimental.pallas import tpu as pltpu
from jax.experimental.pallas import tpu_sc as plsc

N_NODES = 10000
N_EDGES = 160000
D_IN = 256
D_OUT = 256

NC = 2
NS = 16
Q = 64
CW = 16

BLK = 128
EPT = 10240
NBLK = EPT // BLK
NPAIR = NBLK // 2
EDGES_PAD = EPT * NS
N_GARB = 240
NACC = N_NODES + N_GARB

ROWS_PER_TILE = NACC // NS
ZROWS = 160


def _sc_segment_sum(x4, srcs, dsts):
    mesh = plsc.VectorSubcoreMesh(core_axis_name="c", subcore_axis_name="s")

    @functools.partial(
        pl.kernel,
        out_type=(jax.ShapeDtypeStruct((NC, NACC, 2 * Q), jnp.float32),
                  jax.ShapeDtypeStruct((NACC, CW), jnp.float32)),
        mesh=mesh,
        scratch_types=[
            pltpu.VMEM((NBLK, BLK), jnp.int32),
            pltpu.VMEM((NBLK, BLK), jnp.int32),
            pltpu.VMEM((NBLK, BLK), jnp.int32),
            pltpu.VMEM((BLK, Q), jnp.float32),
            pltpu.VMEM((BLK, Q), jnp.float32),
            pltpu.VMEM((ZROWS, Q), jnp.float32),
            pltpu.VMEM((ZROWS, CW), jnp.float32),
            pltpu.VMEM((BLK, CW), jnp.float32),
            pltpu.VMEM_SHARED((NACC, Q), jnp.float32),
            pltpu.VMEM_SHARED((NACC, CW), jnp.float32),
            pltpu.SemaphoreType.DMA,
            pltpu.SemaphoreType.DMA,
            pltpu.SemaphoreType.DMA,
            pltpu.SemaphoreType.DMA,
            pltpu.SemaphoreType.DMA,
            pltpu.SemaphoreType.DMA,
        ],
        compiler_params=pltpu.CompilerParams(use_tc_tiling_on_sc=False,
                                             vmem_limit_bytes=2 * 1024 * 1024),
    )
    def seg_sum(x4_hbm, srcs_hbm, dsts_hbm, out_hbm, outc_hbm,
                sidx0, sidx1, didx, buf0, buf1, zbuf, zbufc, obuf,
                acc, cacc, g0, g1, s0, s1, sc, sz):
        cid = lax.axis_index("c")
        tid = lax.axis_index("s")

        pltpu.async_copy(srcs_hbm.at[cid, 0, tid], sidx0, sz)
        pltpu.async_copy(srcs_hbm.at[cid, 1, tid], sidx1, sz)
        pltpu.async_copy(dsts_hbm.at[tid], didx, sz)

        zero16 = jnp.zeros((16,), jnp.float32)
        one16 = jnp.ones((16,), jnp.float32)

        def fill_zeros(i, _):
            for j in range(Q // 16):
                zbuf[i, pl.ds(j * 16, 16)] = zero16
            zbufc[i, pl.ds(0, 16)] = zero16
            return 0
        lax.fori_loop(0, ZROWS, fill_zeros, 0)

        def fill_ones(i, _):
            obuf[i, pl.ds(0, 16)] = one16
            return 0
        lax.fori_loop(0, BLK, fill_ones, 0)

        pltpu.make_async_copy(srcs_hbm.at[cid, 0, tid], sidx0, sz).wait()
        pltpu.make_async_copy(srcs_hbm.at[cid, 1, tid], sidx1, sz).wait()
        pltpu.make_async_copy(dsts_hbm.at[tid], didx, sz).wait()

        def zero_acc():
            for i in range(ROWS_PER_TILE // ZROWS):
                pltpu.async_copy(
                    zbuf, acc.at[pl.ds(tid * ROWS_PER_TILE + i * ZROWS, ZROWS)],
                    sz)
            for i in range(ROWS_PER_TILE // ZROWS):
                pltpu.make_async_copy(
                    zbuf, acc.at[pl.ds(0, ZROWS)], sz).wait()

        def run_pass(sidx, lane_off, with_counts):
            def start_gather(j, buf, sem):
                pltpu.async_copy(x4_hbm.at[sidx.at[j]], buf, sem)

            def wait_gather(buf, sem):
                pltpu.make_async_copy(x4_hbm.at[pl.ds(0, BLK)], buf, sem).wait()

            def start_scatter(j, buf, sem):
                pltpu.async_copy(buf, acc.at[didx.at[j]], sem, add=True)

            def wait_scatter(buf, sem):
                pltpu.make_async_copy(buf, acc.at[pl.ds(0, BLK)], sem).wait()

            def count_block(j):
                if with_counts:
                    @pl.when(cid == 0)
                    def _():
                        pltpu.async_copy(obuf, cacc.at[didx.at[j]], sc,
                                         add=True)

            def count_drain():
                if with_counts:
                    @pl.when(cid == 0)
                    def _():
                        pltpu.make_async_copy(
                            obuf, cacc.at[pl.ds(0, BLK)], sc).wait()

            start_gather(0, buf0, g0)

            def pair(i, _):
                wait_gather(buf0, g0)

                @pl.when(i > 0)
                def _():
                    wait_scatter(buf1, s1)
                    count_drain()
                    count_drain()

                start_gather(2 * i + 1, buf1, g1)
                start_scatter(2 * i, buf0, s0)
                count_block(2 * i)
                wait_gather(buf1, g1)
                wait_scatter(buf0, s0)

                @pl.when(i < NPAIR - 1)
                def _():
                    start_gather(2 * i + 2, buf0, g0)

                start_scatter(2 * i + 1, buf1, s1)
                count_block(2 * i + 1)
                return 0

            lax.fori_loop(0, NPAIR, pair, 0)
            wait_scatter(buf1, s1)
            count_drain()
            count_drain()
            plsc.subcore_barrier()

            pltpu.sync_copy(
                acc.at[pl.ds(tid * ROWS_PER_TILE, ROWS_PER_TILE)],
                out_hbm.at[cid, pl.ds(tid * ROWS_PER_TILE, ROWS_PER_TILE),
                           pl.ds(lane_off, Q)])

        zero_acc()

        @pl.when(cid == 0)
        def _():
            for i in range(ROWS_PER_TILE // ZROWS):
                pltpu.async_copy(
                    zbufc,
                    cacc.at[pl.ds(tid * ROWS_PER_TILE + i * ZROWS, ZROWS)],
                    sz)
            for i in range(ROWS_PER_TILE // ZROWS):
                pltpu.make_async_copy(
                    zbufc, cacc.at[pl.ds(0, ZROWS)], sz).wait()

        plsc.subcore_barrier()
        run_pass(sidx0, 0, with_counts=True)

        @pl.when(cid == 0)
        def _():
            pltpu.sync_copy(
                cacc.at[pl.ds(tid * ROWS_PER_TILE, ROWS_PER_TILE)],
                outc_hbm.at[pl.ds(tid * ROWS_PER_TILE, ROWS_PER_TILE)])

        zero_acc()
        plsc.subcore_barrier()
        run_pass(sidx1, Q, with_counts=False)

    return seg_sum(x4, srcs, dsts)


def _tc_linear(x, sums, cnt, W, b2, block_m=2000):
    m_blocks = N_NODES // block_m

    def body(x_ref, s0_ref, s1_ref, c_ref, w_ref, b_ref, o_ref):
        cnt_col = jnp.maximum(c_ref[:, 0:1], 1.0)
        agg = jnp.concatenate([s0_ref[0], s1_ref[0]], axis=1) / cnt_col
        w = w_ref[...]
        dn = (((1,), (1,)), ((), ()))
        acc = lax.dot_general(x_ref[...], w[:, :D_IN], dn,
                              preferred_element_type=jnp.float32)
        acc += lax.dot_general(agg, w[:, D_IN:], dn,
                               preferred_element_type=jnp.float32)
        o_ref[...] = acc + b_ref[...]

    return pl.pallas_call(
        body,
        grid=(m_blocks,),
        in_specs=[
            pl.BlockSpec((block_m, D_IN), lambda m: (m, 0)),
            pl.BlockSpec((1, block_m, 2 * Q), lambda m: (0, m, 0)),
            pl.BlockSpec((1, block_m, 2 * Q), lambda m: (1, m, 0)),
            pl.BlockSpec((block_m, CW), lambda m: (m, 0)),
            pl.BlockSpec((D_OUT, 2 * D_IN), lambda m: (0, 0)),
            pl.BlockSpec((1, D_OUT), lambda m: (0, 0)),
        ],
        out_specs=pl.BlockSpec((block_m, D_OUT), lambda m: (m, 0)),
        out_shape=jax.ShapeDtypeStruct((N_NODES, D_OUT), jnp.float32),
        compiler_params=pltpu.CompilerParams(
            dimension_semantics=("arbitrary",)),
    )(x, sums, sums, cnt, W, b2)


def kernel(x, edge_index, W, b):
    src = edge_index[0].astype(jnp.int32)
    dst = edge_index[1].astype(jnp.int32)

    npad = EDGES_PAD - N_EDGES
    pad_ids = lax.iota(jnp.int32, npad)
    psrc = jnp.concatenate([src, pad_ids % N_NODES])
    pdst = jnp.concatenate([dst, N_NODES + pad_ids % N_GARB])
    psrc4 = psrc * 4
    srcs = jnp.stack([psrc4, psrc4 + 1, psrc4 + 2, psrc4 + 3])
    srcs = srcs.reshape(NC, 2, EDGES_PAD).reshape(NC, 2, NS, NBLK, BLK)
    dsts = pdst.reshape(NS, NBLK, BLK)

    x4 = x.reshape(4 * N_NODES, Q)

    sums, cnt = _sc_segment_sum(x4, srcs, dsts)
    return _tc_linear(x, sums, cnt, W, b.reshape(1, D_OUT))
